# baseline (speedup 1.0000x reference)
import functools

import jax
import jax.numpy as jnp
from jax import lax
from jax.experimental import pallas as pl
from jax.experimental.pallas import tpu as pltpu
from jax.experimental.pallas import tpu_sc as plsc

N = 50000
E = 1600000
G = 128
NCORES = 2
NSUB = 16
NTILES = NCORES * NSUB
ROWS_PER_TILE = 1568
NPAD = NTILES * ROWS_PER_TILE
EDGES_PER_TILE = E // NTILES
CHUNK = 400
NB = 400
NGRID = N // NB
PCHUNK = 224

_SC_MESH = plsc.VectorSubcoreMesh(core_axis_name="c", subcore_axis_name="s")
_SC_PARAMS = pltpu.CompilerParams(
    use_tc_tiling_on_sc=False, needs_layout_passes=False)


def _deg_body(msg8_hbm, dst_hbm, zero8_hbm, out_hbm, idx_v, msg_v, acc):
    c = lax.axis_index("c")
    s = lax.axis_index("s")
    row0 = s * ROWS_PER_TILE
    pltpu.sync_copy(zero8_hbm, acc.at[pl.ds(row0, ROWS_PER_TILE)])
    plsc.subcore_barrier()
    base = (c * NSUB + s) * EDGES_PER_TILE

    def body(i, carry):
        off = base + i * CHUNK
        pltpu.sync_copy(dst_hbm.at[pl.ds(off, CHUNK)], idx_v)
        pltpu.sync_copy(msg8_hbm.at[pl.ds(off, CHUNK)], msg_v)
        pltpu.sync_copy(msg_v, acc.at[idx_v], add=True)
        return carry

    lax.fori_loop(0, EDGES_PER_TILE // CHUNK, body, 0)
    plsc.subcore_barrier()
    pltpu.sync_copy(acc.at[pl.ds(row0, ROWS_PER_TILE)],
                    out_hbm.at[c, pl.ds(row0, ROWS_PER_TILE)])


_deg_kernel = functools.partial(
    pl.kernel,
    out_type=jax.ShapeDtypeStruct((NCORES, NPAD, 8), jnp.float32),
    mesh=_SC_MESH,
    scratch_types=[
        pltpu.VMEM((CHUNK,), jnp.int32),
        pltpu.VMEM((CHUNK, 8), jnp.float32),
        pltpu.VMEM_SHARED((NPAD, 8), jnp.float32),
    ],
    compiler_params=_SC_PARAMS,
)(_deg_body)


def _conv1_body(xp_hbm, src_hbm, dst_hbm, zeros_hbm, out_hbm,
                src_v, dst_v, rows_v, acc):
    c = lax.axis_index("c")
    s = lax.axis_index("s")
    row0 = s * ROWS_PER_TILE
    pltpu.sync_copy(zeros_hbm, acc.at[pl.ds(row0, ROWS_PER_TILE)])
    plsc.subcore_barrier()
    base = (c * NSUB + s) * EDGES_PER_TILE

    def body(i, carry):
        off = base + i * CHUNK
        pltpu.sync_copy(src_hbm.at[pl.ds(off, CHUNK)], src_v)
        pltpu.sync_copy(dst_hbm.at[pl.ds(off, CHUNK)], dst_v)
        pltpu.sync_copy(xp_hbm.at[src_v], rows_v)
        pltpu.sync_copy(rows_v, acc.at[dst_v], add=True)
        return carry

    lax.fori_loop(0, EDGES_PER_TILE // CHUNK, body, 0)
    plsc.subcore_barrier()
    pltpu.sync_copy(acc.at[pl.ds(row0, ROWS_PER_TILE)],
                    out_hbm.at[c, pl.ds(row0, ROWS_PER_TILE)])


_conv1_scatter = functools.partial(
    pl.kernel,
    out_type=jax.ShapeDtypeStruct((NCORES, NPAD, 32), jnp.float32),
    mesh=_SC_MESH,
    scratch_types=[
        pltpu.VMEM((CHUNK,), jnp.int32),
        pltpu.VMEM((CHUNK,), jnp.int32),
        pltpu.VMEM((CHUNK, 32), jnp.float32),
        pltpu.VMEM_SHARED((NPAD, 32), jnp.float32),
    ],
    compiler_params=_SC_PARAMS,
)(_conv1_body)


def _conv2_body(hp_hbm, src_hbm, dst_hbm, ew16_hbm, zeros_hbm, out_hbm,
                src_v, dst_v, ew16_v, rows_v, acc):
    c = lax.axis_index("c")
    s = lax.axis_index("s")
    row0 = s * ROWS_PER_TILE
    base = (c * NSUB + s) * EDGES_PER_TILE
    for k in range(4):
        pltpu.sync_copy(zeros_hbm, acc.at[pl.ds(row0, ROWS_PER_TILE)])
        plsc.subcore_barrier()

        def body(i, carry):
            off = base + i * CHUNK
            pltpu.sync_copy(src_hbm.at[pl.ds(off, CHUNK)], src_v)
            pltpu.sync_copy(dst_hbm.at[pl.ds(off, CHUNK)], dst_v)
            pltpu.sync_copy(ew16_hbm.at[pl.ds(off, CHUNK)], ew16_v)
            pltpu.sync_copy(hp_hbm.at[k].at[src_v], rows_v)

            def grp(g, c2):
                e0 = g * 16
                for j in range(16):
                    e = e0 + j
                    wv = ew16_v[e, pl.ds(0, 16)]
                    rows_v[e, pl.ds(0, 16)] = rows_v[e, pl.ds(0, 16)] * wv
                    rows_v[e, pl.ds(16, 16)] = rows_v[e, pl.ds(16, 16)] * wv
                return c2

            lax.fori_loop(0, CHUNK // 16, grp, 0)
            pltpu.sync_copy(rows_v, acc.at[dst_v], add=True)
            return carry

        lax.fori_loop(0, EDGES_PER_TILE // CHUNK, body, 0)
        plsc.subcore_barrier()
        pltpu.sync_copy(acc.at[pl.ds(row0, ROWS_PER_TILE)],
                        out_hbm.at[c, k, pl.ds(row0, ROWS_PER_TILE)])
        plsc.subcore_barrier()


_conv2_scatter = functools.partial(
    pl.kernel,
    out_type=jax.ShapeDtypeStruct((NCORES, 4, NPAD, 32), jnp.float32),
    mesh=_SC_MESH,
    scratch_types=[
        pltpu.VMEM((CHUNK,), jnp.int32),
        pltpu.VMEM((CHUNK,), jnp.int32),
        pltpu.VMEM((CHUNK, 16), jnp.float32),
        pltpu.VMEM((CHUNK, 32), jnp.float32),
        pltpu.VMEM_SHARED((NPAD, 32), jnp.float32),
    ],
    compiler_params=_SC_PARAMS,
)(_conv2_body)


def _pool_body(p_hbm, bat_hbm, out_hbm, pbuf, bat_v, amax, amin):
    c = lax.axis_index("c")
    s = lax.axis_index("s")
    wid = c * NSUB + s
    ninf = jnp.full((16,), -jnp.inf, jnp.float32)
    pinf = jnp.full((16,), jnp.inf, jnp.float32)

    def init(i, carry):
        amax[pl.ds(i * 16, 16)] = ninf
        amin[pl.ds(i * 16, 16)] = pinf
        return carry

    lax.fori_loop(0, G * 128 // 16, init, 0)
    base = wid * ROWS_PER_TILE
    lanes = jnp.arange(16, dtype=jnp.int32)

    def chunk_body(cidx, carry):
        r0 = base + cidx * PCHUNK
        pltpu.sync_copy(p_hbm.at[pl.ds(r0, PCHUNK)], pbuf)
        pltpu.sync_copy(bat_hbm.at[pl.ds(r0, PCHUNK)], bat_v)

        def grp_body(gi, c2):
            g0 = gi * 16
            bat16 = bat_v[pl.ds(g0, 16)]
            for j in range(16):
                r = g0 + j
                gid = jnp.sum(jnp.where(lanes == j, bat16, 0))

                @pl.when(r0 + r < N)
                def _():
                    tb = gid * 128
                    for fb in range(8):
                        sl = pl.ds(tb + fb * 16, 16)
                        val = pbuf[r, pl.ds(fb * 16, 16)]
                        amax[sl] = jnp.maximum(amax[sl], val)
                        amin[sl] = jnp.minimum(amin[sl], val)
            return c2

        lax.fori_loop(0, PCHUNK // 16, grp_body, 0)
        return carry

    lax.fori_loop(0, ROWS_PER_TILE // PCHUNK, chunk_body, 0)
    pltpu.sync_copy(amax, out_hbm.at[wid, 0])
    pltpu.sync_copy(amin, out_hbm.at[wid, 1])


_pool_kernel = functools.partial(
    pl.kernel,
    out_type=jax.ShapeDtypeStruct((NTILES, 2, G * 128), jnp.float32),
    mesh=_SC_MESH,
    scratch_types=[
        pltpu.VMEM((PCHUNK, 128), jnp.float32),
        pltpu.VMEM((PCHUNK,), jnp.int32),
        pltpu.VMEM((G * 128,), jnp.float32),
        pltpu.VMEM((G * 128,), jnp.float32),
    ],
    compiler_params=_SC_PARAMS,
)(_pool_body)


def _t1_body(deg8_ref, x_ref, dis1_ref, dis2_ref, xp_ref):
    d = deg8_ref[0] + deg8_ref[1]
    r1 = lax.rsqrt(d[:, 0:1] + 1.0)
    r2 = lax.rsqrt(d[:, 1:2] + 1.0)
    dis1_ref[...] = r1
    dis2_ref[...] = r2
    xp_ref[...] = x_ref[...] * r1


_t1 = pl.pallas_call(
    _t1_body,
    grid=(NGRID,),
    in_specs=[
        pl.BlockSpec((2, NB, 8), lambda i: (0, i, 0)),
        pl.BlockSpec((NB, 32), lambda i: (i, 0)),
    ],
    out_specs=[
        pl.BlockSpec((NB, 1), lambda i: (i, 0)),
        pl.BlockSpec((NB, 1), lambda i: (i, 0)),
        pl.BlockSpec((NB, 32), lambda i: (i, 0)),
    ],
    out_shape=[
        jax.ShapeDtypeStruct((N, 1), jnp.float32),
        jax.ShapeDtypeStruct((N, 1), jnp.float32),
        jax.ShapeDtypeStruct((N, 32), jnp.float32),
    ],
)


def _onehot(bat):
    return (bat == lax.broadcasted_iota(jnp.int32, (1, G), 1)).astype(jnp.float32)


def _t2_body(s1_ref, xp_ref, dis1_ref, batch_ref, W1_ref, b1_ref,
             h_ref, S1a_ref, S1b_ref, cnt_ref):
    i = pl.program_id(0)
    agg = s1_ref[0] + s1_ref[1] + xp_ref[...]
    pre = dis1_ref[...] * agg
    h = jnp.dot(pre, W1_ref[...], preferred_element_type=jnp.float32) + b1_ref[...]
    h_ref[...] = h
    M = _onehot(batch_ref[...])
    dn = (((0,), (0,)), ((), ()))
    sa = lax.dot_general(M, h, dn, preferred_element_type=jnp.float32)
    sb = lax.dot_general(M, h * h, dn, preferred_element_type=jnp.float32)
    cl = jnp.sum(M, axis=0, keepdims=True)

    @pl.when(i == 0)
    def _():
        S1a_ref[...] = sa
        S1b_ref[...] = sb
        cnt_ref[...] = cl

    @pl.when(i != 0)
    def _():
        S1a_ref[...] += sa
        S1b_ref[...] += sb
        cnt_ref[...] += cl


_t2 = pl.pallas_call(
    _t2_body,
    grid=(NGRID,),
    in_specs=[
        pl.BlockSpec((2, NB, 32), lambda i: (0, i, 0)),
        pl.BlockSpec((NB, 32), lambda i: (i, 0)),
        pl.BlockSpec((NB, 1), lambda i: (i, 0)),
        pl.BlockSpec((NB, 1), lambda i: (i, 0)),
        pl.BlockSpec((32, G), lambda i: (0, 0)),
        pl.BlockSpec((1, G), lambda i: (0, 0)),
    ],
    out_specs=[
        pl.BlockSpec((NB, G), lambda i: (i, 0)),
        pl.BlockSpec((G, G), lambda i: (0, 0)),
        pl.BlockSpec((G, G), lambda i: (0, 0)),
        pl.BlockSpec((1, G), lambda i: (0, 0)),
    ],
    out_shape=[
        jax.ShapeDtypeStruct((N, G), jnp.float32),
        jax.ShapeDtypeStruct((G, G), jnp.float32),
        jax.ShapeDtypeStruct((G, G), jnp.float32),
        jax.ShapeDtypeStruct((1, G), jnp.float32),
    ],
)


def _t2b_body(h_ref, S1a_ref, S1b_ref, cnt_ref, dis2_ref, batch_ref,
              w_ref, b_ref, s_ref, w1_ref, hp4_ref):
    cl = jnp.maximum(cnt_ref[...], 1.0)
    invc = jnp.reshape(1.0 / cl, (G, 1))
    mean = S1a_ref[...] * invc
    ex2 = S1b_ref[...] * invc
    s = s_ref[...]
    var = ex2 - mean * mean * s * (2.0 - s)
    std = jnp.sqrt(var + 1e-5)
    M = _onehot(batch_ref[...])
    mean_row = jnp.dot(M, mean, preferred_element_type=jnp.float32)
    std_row = jnp.dot(M, std, preferred_element_type=jnp.float32)
    h = h_ref[...]
    g1 = w_ref[...] * (h - mean_row * s) / std_row + b_ref[...]
    p = jnp.where(g1 >= 0, g1, g1 * w1_ref[...])
    hp = p * dis2_ref[...]
    hp4_ref[...] = jnp.stack(
        [hp[:, 0:32], hp[:, 32:64], hp[:, 64:96], hp[:, 96:128]])


_t2b = pl.pallas_call(
    _t2b_body,
    grid=(NGRID,),
    in_specs=[
        pl.BlockSpec((NB, G), lambda i: (i, 0)),
        pl.BlockSpec((G, G), lambda i: (0, 0)),
        pl.BlockSpec((G, G), lambda i: (0, 0)),
        pl.BlockSpec((1, G), lambda i: (0, 0)),
        pl.BlockSpec((NB, 1), lambda i: (i, 0)),
        pl.BlockSpec((NB, 1), lambda i: (i, 0)),
        pl.BlockSpec((1, G), lambda i: (0, 0)),
        pl.BlockSpec((1, G), lambda i: (0, 0)),
        pl.BlockSpec((1, G), lambda i: (0, 0)),
        pl.BlockSpec((1, G), lambda i: (0, 0)),
    ],
    out_specs=[pl.BlockSpec((4, NB, 32), lambda i: (0, i, 0))],
    out_shape=[jax.ShapeDtypeStruct((4, N, 32), jnp.float32)],
)


def _t3_body(s2_ref, hp4_ref, dis2_ref, batch_ref, W2_ref, b2_ref, w1_ref,
             p_ref, S2a_ref, S2b_ref):
    i = pl.program_id(0)
    agg = jnp.concatenate(
        [s2_ref[0, k] + s2_ref[1, k] + hp4_ref[k] for k in range(4)], axis=1)
    pre = dis2_ref[...] * agg
    h2 = jnp.dot(pre, W2_ref[...], preferred_element_type=jnp.float32) + b2_ref[...]
    p = jnp.where(h2 >= 0, h2, h2 * w1_ref[...])
    p_ref[...] = p
    M = _onehot(batch_ref[...])
    dn = (((0,), (0,)), ((), ()))
    sa = lax.dot_general(M, p, dn, preferred_element_type=jnp.float32)
    sb = lax.dot_general(M, p * p, dn, preferred_element_type=jnp.float32)

    @pl.when(i == 0)
    def _():
        S2a_ref[...] = sa
        S2b_ref[...] = sb

    @pl.when(i != 0)
    def _():
        S2a_ref[...] += sa
        S2b_ref[...] += sb


_t3 = pl.pallas_call(
    _t3_body,
    grid=(NGRID,),
    in_specs=[
        pl.BlockSpec((2, 4, NB, 32), lambda i: (0, 0, i, 0)),
        pl.BlockSpec((4, NB, 32), lambda i: (0, i, 0)),
        pl.BlockSpec((NB, 1), lambda i: (i, 0)),
        pl.BlockSpec((NB, 1), lambda i: (i, 0)),
        pl.BlockSpec((G, G), lambda i: (0, 0)),
        pl.BlockSpec((1, G), lambda i: (0, 0)),
        pl.BlockSpec((1, G), lambda i: (0, 0)),
    ],
    out_specs=[
        pl.BlockSpec((NB, G), lambda i: (i, 0)),
        pl.BlockSpec((G, G), lambda i: (0, 0)),
        pl.BlockSpec((G, G), lambda i: (0, 0)),
    ],
    out_shape=[
        jax.ShapeDtypeStruct((NPAD, G), jnp.float32),
        jax.ShapeDtypeStruct((G, G), jnp.float32),
        jax.ShapeDtypeStruct((G, G), jnp.float32),
    ],
)


def _t4_body(mm_ref, acc_ref):
    i = pl.program_id(0)
    blk = mm_ref[0]

    @pl.when(i == 0)
    def _():
        acc_ref[...] = blk

    @pl.when(i != 0)
    def _():
        cur = acc_ref[...]
        acc_ref[...] = jnp.concatenate(
            [jnp.maximum(cur[0:G], blk[0:G]),
             jnp.minimum(cur[G:2 * G], blk[G:2 * G])], axis=0)


_t4 = pl.pallas_call(
    _t4_body,
    grid=(NTILES,),
    in_specs=[pl.BlockSpec((1, 2 * G, 128), lambda i: (i, 0, 0))],
    out_specs=[pl.BlockSpec((2 * G, 128), lambda i: (0, 0))],
    out_shape=[jax.ShapeDtypeStruct((2 * G, 128), jnp.float32)],
)


def _t5_body(S2a_ref, S2b_ref, cnt_ref, mm_ref, w_ref, b_ref, s_ref,
             w1_ref, w2_ref, l1W_ref, l1b_ref, l2W_ref, l2b_ref,
             l3W_ref, l3b_ref, out_ref):
    cl = jnp.maximum(cnt_ref[...], 1.0)
    invc = jnp.reshape(1.0 / cl, (G, 1))
    meanp = S2a_ref[...] * invc
    ex2 = S2b_ref[...] * invc
    s = s_ref[...]
    var = ex2 - meanp * meanp * s * (2.0 - s)
    std = jnp.sqrt(var + 1e-5)
    alpha = w_ref[...] / std
    beta = b_ref[...] - alpha * meanp * s
    pmax = mm_ref[0:G]
    pmin = mm_ref[G:2 * G]
    x1 = alpha * meanp + beta
    x2 = jnp.where(alpha >= 0, alpha * pmax, alpha * pmin) + beta
    z = jnp.concatenate([x1, x2], axis=1)
    z = jnp.dot(z, l1W_ref[...], preferred_element_type=jnp.float32) + l1b_ref[...]
    z = jnp.where(z >= 0, z, z * w1_ref[...])
    z = jnp.dot(z, l2W_ref[...], preferred_element_type=jnp.float32) + l2b_ref[...]
    z = jnp.where(z >= 0, z, z * w2_ref[...])
    z = jnp.dot(z, l3W_ref[...], preferred_element_type=jnp.float32) + l3b_ref[...]
    out_ref[...] = z


_t5 = pl.pallas_call(
    _t5_body,
    grid=(1,),
    in_specs=[
        pl.BlockSpec((G, G), lambda i: (0, 0)),
        pl.BlockSpec((G, G), lambda i: (0, 0)),
        pl.BlockSpec((1, G), lambda i: (0, 0)),
        pl.BlockSpec((2 * G, 128), lambda i: (0, 0)),
        pl.BlockSpec((1, G), lambda i: (0, 0)),
        pl.BlockSpec((1, G), lambda i: (0, 0)),
        pl.BlockSpec((1, G), lambda i: (0, 0)),
        pl.BlockSpec((1, G), lambda i: (0, 0)),
        pl.BlockSpec((1, 64), lambda i: (0, 0)),
        pl.BlockSpec((2 * G, G), lambda i: (0, 0)),
        pl.BlockSpec((1, G), lambda i: (0, 0)),
        pl.BlockSpec((G, 64), lambda i: (0, 0)),
        pl.BlockSpec((1, 64), lambda i: (0, 0)),
        pl.BlockSpec((64, 1), lambda i: (0, 0)),
        pl.BlockSpec((1, 1), lambda i: (0, 0)),
    ],
    out_specs=[pl.BlockSpec((G, 1), lambda i: (0, 0))],
    out_shape=[jax.ShapeDtypeStruct((G, 1), jnp.float32)],
)


def kernel(x, edge_index, edge_attr, batch, weight1, weight2, conv1_W, conv1_b,
           conv2_W, conv2_b, lin1_W, lin1_b, lin2_W, lin2_b, lin3_W, lin3_b,
           bn1_weight, bn1_bias, bn1_mean_scale, bn2_weight, bn2_bias,
           bn2_mean_scale):
    src = edge_index[0]
    dst = edge_index[1]
    batch2d = batch[:, None]
    batch_pad = jnp.pad(batch, (0, NPAD - N))

    zeros8 = jnp.zeros((ROWS_PER_TILE, 8), jnp.float32)
    zeros32 = jnp.zeros((ROWS_PER_TILE, 32), jnp.float32)
    msg8 = jnp.concatenate(
        [jnp.ones((E, 1), jnp.float32), edge_attr[:, None],
         jnp.zeros((E, 6), jnp.float32)], axis=1)

    deg8 = _deg_kernel(msg8, dst, zeros8)
    dis1, dis2, xp = _t1(deg8, x)
    s1 = _conv1_scatter(xp, src, dst, zeros32)
    h, S1a, S1b, cnt = _t2(s1, xp, dis1, batch2d, conv1_W, conv1_b[None, :])
    (hp4,) = _t2b(h, S1a, S1b, cnt, dis2, batch2d, bn1_weight[None, :],
                  bn1_bias[None, :], bn1_mean_scale[None, :], weight1[None, :])
    ew16 = jnp.broadcast_to(edge_attr[:, None], (E, 16))
    s2 = _conv2_scatter(hp4, src, dst, ew16, zeros32)
    p, S2a, S2b = _t3(s2, hp4, dis2, batch2d, conv2_W, conv2_b[None, :],
                      weight1[None, :])
    mm = _pool_kernel(p, batch_pad)
    (mm2,) = _t4(mm.reshape(NTILES, 2 * G, 128))
    (out,) = _t5(S2a, S2b, cnt, mm2, bn2_weight[None, :], bn2_bias[None, :],
                 bn2_mean_scale[None, :], weight1[None, :], weight2[None, :],
                 lin1_W, lin1_b[None, :], lin2_W, lin2_b[None, :], lin3_W,
                 lin3_b[None, :])
    return jnp.squeeze(out)


# speedup vs baseline: 1.1543x; 1.1543x over previous
import functools

import jax
import jax.numpy as jnp
from jax import lax
from jax.experimental import pallas as pl
from jax.experimental.pallas import tpu as pltpu
from jax.experimental.pallas import tpu_sc as plsc

N = 50000
E = 1600000
G = 128
NCORES = 2
NSUB = 16
NTILES = NCORES * NSUB
ROWS_PER_TILE = 1568
NPAD = NTILES * ROWS_PER_TILE
EDGES_PER_TILE = E // NTILES
CHUNK = 400
NB = 400
NGRID = N // NB
PCHUNK = 224

_SC_MESH = plsc.VectorSubcoreMesh(core_axis_name="c", subcore_axis_name="s")
_SC_PARAMS = pltpu.CompilerParams(
    use_tc_tiling_on_sc=False, needs_layout_passes=False)


def _deg_body(msg8_hbm, dst_hbm, zero8_hbm, out_hbm, idx_v, msg_v, acc):
    c = lax.axis_index("c")
    s = lax.axis_index("s")
    row0 = s * ROWS_PER_TILE
    pltpu.sync_copy(zero8_hbm, acc.at[pl.ds(row0, ROWS_PER_TILE)])
    plsc.subcore_barrier()
    base = (c * NSUB + s) * EDGES_PER_TILE

    def body(i, carry):
        off = base + i * CHUNK
        pltpu.sync_copy(dst_hbm.at[pl.ds(off, CHUNK)], idx_v)
        pltpu.sync_copy(msg8_hbm.at[pl.ds(off, CHUNK)], msg_v)
        pltpu.sync_copy(msg_v, acc.at[idx_v], add=True)
        return carry

    lax.fori_loop(0, EDGES_PER_TILE // CHUNK, body, 0)
    plsc.subcore_barrier()
    pltpu.sync_copy(acc.at[pl.ds(row0, ROWS_PER_TILE)],
                    out_hbm.at[c, pl.ds(row0, ROWS_PER_TILE)])


_deg_kernel = functools.partial(
    pl.kernel,
    out_type=jax.ShapeDtypeStruct((NCORES, NPAD, 8), jnp.float32),
    mesh=_SC_MESH,
    scratch_types=[
        pltpu.VMEM((CHUNK,), jnp.int32),
        pltpu.VMEM((CHUNK, 8), jnp.float32),
        pltpu.VMEM_SHARED((NPAD, 8), jnp.float32),
    ],
    compiler_params=_SC_PARAMS,
)(_deg_body)


def _conv1_body(xp_hbm, src_hbm, dst_hbm, zeros_hbm, out_hbm,
                src_v, dst_v, rows_v, acc):
    c = lax.axis_index("c")
    s = lax.axis_index("s")
    row0 = s * ROWS_PER_TILE
    pltpu.sync_copy(zeros_hbm, acc.at[pl.ds(row0, ROWS_PER_TILE)])
    plsc.subcore_barrier()
    base = (c * NSUB + s) * EDGES_PER_TILE

    def body(i, carry):
        off = base + i * CHUNK
        pltpu.sync_copy(src_hbm.at[pl.ds(off, CHUNK)], src_v)
        pltpu.sync_copy(dst_hbm.at[pl.ds(off, CHUNK)], dst_v)
        pltpu.sync_copy(xp_hbm.at[src_v], rows_v)
        pltpu.sync_copy(rows_v, acc.at[dst_v], add=True)
        return carry

    lax.fori_loop(0, EDGES_PER_TILE // CHUNK, body, 0)
    plsc.subcore_barrier()
    pltpu.sync_copy(acc.at[pl.ds(row0, ROWS_PER_TILE)],
                    out_hbm.at[c, pl.ds(row0, ROWS_PER_TILE)])


_conv1_scatter = functools.partial(
    pl.kernel,
    out_type=jax.ShapeDtypeStruct((NCORES, NPAD, 32), jnp.float32),
    mesh=_SC_MESH,
    scratch_types=[
        pltpu.VMEM((CHUNK,), jnp.int32),
        pltpu.VMEM((CHUNK,), jnp.int32),
        pltpu.VMEM((CHUNK, 32), jnp.float32),
        pltpu.VMEM_SHARED((NPAD, 32), jnp.float32),
    ],
    compiler_params=_SC_PARAMS,
)(_conv1_body)


def _conv2_body(hp_hbm, src_hbm, dst_hbm, ew_hbm, zeros_hbm, out_hbm,
                src_v, dst_v, ew_v, rows_v, acc):
    c = lax.axis_index("c")
    s = lax.axis_index("s")
    row0 = s * ROWS_PER_TILE
    base = (c * NSUB + s) * EDGES_PER_TILE
    lanes = jnp.arange(16, dtype=jnp.int32)
    for k in range(4):
        pltpu.sync_copy(zeros_hbm, acc.at[pl.ds(row0, ROWS_PER_TILE)])
        plsc.subcore_barrier()

        def body(i, carry):
            off = base + i * CHUNK
            pltpu.sync_copy(src_hbm.at[pl.ds(off, CHUNK)], src_v)
            pltpu.sync_copy(dst_hbm.at[pl.ds(off, CHUNK)], dst_v)
            pltpu.sync_copy(ew_hbm.at[pl.ds(off, CHUNK)], ew_v)
            pltpu.sync_copy(hp_hbm.at[k].at[src_v], rows_v)

            def grp(g, c2):
                e0 = g * 16
                ew16 = ew_v[pl.ds(e0, 16)]
                for j in range(16):
                    e = e0 + j
                    w = jnp.sum(jnp.where(lanes == j, ew16, 0.0))
                    rows_v[e, pl.ds(0, 16)] = rows_v[e, pl.ds(0, 16)] * w
                    rows_v[e, pl.ds(16, 16)] = rows_v[e, pl.ds(16, 16)] * w
                return c2

            lax.fori_loop(0, CHUNK // 16, grp, 0)
            pltpu.sync_copy(rows_v, acc.at[dst_v], add=True)
            return carry

        lax.fori_loop(0, EDGES_PER_TILE // CHUNK, body, 0)
        plsc.subcore_barrier()
        pltpu.sync_copy(acc.at[pl.ds(row0, ROWS_PER_TILE)],
                        out_hbm.at[c, k, pl.ds(row0, ROWS_PER_TILE)])
        plsc.subcore_barrier()


_conv2_scatter = functools.partial(
    pl.kernel,
    out_type=jax.ShapeDtypeStruct((NCORES, 4, NPAD, 32), jnp.float32),
    mesh=_SC_MESH,
    scratch_types=[
        pltpu.VMEM((CHUNK,), jnp.int32),
        pltpu.VMEM((CHUNK,), jnp.int32),
        pltpu.VMEM((CHUNK,), jnp.float32),
        pltpu.VMEM((CHUNK, 32), jnp.float32),
        pltpu.VMEM_SHARED((NPAD, 32), jnp.float32),
    ],
    compiler_params=_SC_PARAMS,
)(_conv2_body)


def _pool_body(p_hbm, bat_hbm, out_hbm, pbuf, bat_v, amax, amin):
    c = lax.axis_index("c")
    s = lax.axis_index("s")
    wid = c * NSUB + s
    ninf = jnp.full((16,), -jnp.inf, jnp.float32)
    pinf = jnp.full((16,), jnp.inf, jnp.float32)

    def init(i, carry):
        amax[pl.ds(i * 16, 16)] = ninf
        amin[pl.ds(i * 16, 16)] = pinf
        return carry

    lax.fori_loop(0, G * 128 // 16, init, 0)
    base = wid * ROWS_PER_TILE
    lanes = jnp.arange(16, dtype=jnp.int32)

    def chunk_body(cidx, carry):
        r0 = base + cidx * PCHUNK
        pltpu.sync_copy(p_hbm.at[pl.ds(r0, PCHUNK)], pbuf)
        pltpu.sync_copy(bat_hbm.at[pl.ds(r0, PCHUNK)], bat_v)

        def grp_body(gi, c2):
            g0 = gi * 16
            bat16 = bat_v[pl.ds(g0, 16)]
            for j in range(16):
                r = g0 + j
                gid = jnp.sum(jnp.where(lanes == j, bat16, 0))

                @pl.when(r0 + r < N)
                def _():
                    tb = gid * 128
                    for fb in range(8):
                        sl = pl.ds(tb + fb * 16, 16)
                        val = pbuf[r, pl.ds(fb * 16, 16)]
                        amax[sl] = jnp.maximum(amax[sl], val)
                        amin[sl] = jnp.minimum(amin[sl], val)
            return c2

        lax.fori_loop(0, PCHUNK // 16, grp_body, 0)
        return carry

    lax.fori_loop(0, ROWS_PER_TILE // PCHUNK, chunk_body, 0)
    pltpu.sync_copy(amax, out_hbm.at[wid, 0])
    pltpu.sync_copy(amin, out_hbm.at[wid, 1])


_pool_kernel = functools.partial(
    pl.kernel,
    out_type=jax.ShapeDtypeStruct((NTILES, 2, G * 128), jnp.float32),
    mesh=_SC_MESH,
    scratch_types=[
        pltpu.VMEM((PCHUNK, 128), jnp.float32),
        pltpu.VMEM((PCHUNK,), jnp.int32),
        pltpu.VMEM((G * 128,), jnp.float32),
        pltpu.VMEM((G * 128,), jnp.float32),
    ],
    compiler_params=_SC_PARAMS,
)(_pool_body)


def _t1_body(deg8_ref, x_ref, dis1_ref, dis2_ref, xp_ref):
    d = deg8_ref[0] + deg8_ref[1]
    r1 = lax.rsqrt(d[:, 0:1] + 1.0)
    r2 = lax.rsqrt(d[:, 1:2] + 1.0)
    dis1_ref[...] = r1
    dis2_ref[...] = r2
    xp_ref[...] = x_ref[...] * r1


_t1 = pl.pallas_call(
    _t1_body,
    grid=(NGRID,),
    in_specs=[
        pl.BlockSpec((2, NB, 8), lambda i: (0, i, 0)),
        pl.BlockSpec((NB, 32), lambda i: (i, 0)),
    ],
    out_specs=[
        pl.BlockSpec((NB, 1), lambda i: (i, 0)),
        pl.BlockSpec((NB, 1), lambda i: (i, 0)),
        pl.BlockSpec((NB, 32), lambda i: (i, 0)),
    ],
    out_shape=[
        jax.ShapeDtypeStruct((N, 1), jnp.float32),
        jax.ShapeDtypeStruct((N, 1), jnp.float32),
        jax.ShapeDtypeStruct((N, 32), jnp.float32),
    ],
)


def _onehot(bat):
    return (bat == lax.broadcasted_iota(jnp.int32, (1, G), 1)).astype(jnp.float32)


def _t2_body(s1_ref, xp_ref, dis1_ref, batch_ref, W1_ref, b1_ref,
             h_ref, S1a_ref, S1b_ref, cnt_ref):
    i = pl.program_id(0)
    agg = s1_ref[0] + s1_ref[1] + xp_ref[...]
    pre = dis1_ref[...] * agg
    h = jnp.dot(pre, W1_ref[...], preferred_element_type=jnp.float32) + b1_ref[...]
    h_ref[...] = h
    M = _onehot(batch_ref[...])
    dn = (((0,), (0,)), ((), ()))
    sa = lax.dot_general(M, h, dn, preferred_element_type=jnp.float32)
    sb = lax.dot_general(M, h * h, dn, preferred_element_type=jnp.float32)
    cl = jnp.sum(M, axis=0, keepdims=True)

    @pl.when(i == 0)
    def _():
        S1a_ref[...] = sa
        S1b_ref[...] = sb
        cnt_ref[...] = cl

    @pl.when(i != 0)
    def _():
        S1a_ref[...] += sa
        S1b_ref[...] += sb
        cnt_ref[...] += cl


_t2 = pl.pallas_call(
    _t2_body,
    grid=(NGRID,),
    in_specs=[
        pl.BlockSpec((2, NB, 32), lambda i: (0, i, 0)),
        pl.BlockSpec((NB, 32), lambda i: (i, 0)),
        pl.BlockSpec((NB, 1), lambda i: (i, 0)),
        pl.BlockSpec((NB, 1), lambda i: (i, 0)),
        pl.BlockSpec((32, G), lambda i: (0, 0)),
        pl.BlockSpec((1, G), lambda i: (0, 0)),
    ],
    out_specs=[
        pl.BlockSpec((NB, G), lambda i: (i, 0)),
        pl.BlockSpec((G, G), lambda i: (0, 0)),
        pl.BlockSpec((G, G), lambda i: (0, 0)),
        pl.BlockSpec((1, G), lambda i: (0, 0)),
    ],
    out_shape=[
        jax.ShapeDtypeStruct((N, G), jnp.float32),
        jax.ShapeDtypeStruct((G, G), jnp.float32),
        jax.ShapeDtypeStruct((G, G), jnp.float32),
        jax.ShapeDtypeStruct((1, G), jnp.float32),
    ],
)


def _t2b_body(h_ref, S1a_ref, S1b_ref, cnt_ref, dis2_ref, batch_ref,
              w_ref, b_ref, s_ref, w1_ref, hp4_ref):
    cl = jnp.maximum(cnt_ref[...], 1.0)
    invc = jnp.reshape(1.0 / cl, (G, 1))
    mean = S1a_ref[...] * invc
    ex2 = S1b_ref[...] * invc
    s = s_ref[...]
    var = ex2 - mean * mean * s * (2.0 - s)
    std = jnp.sqrt(var + 1e-5)
    M = _onehot(batch_ref[...])
    mean_row = jnp.dot(M, mean, preferred_element_type=jnp.float32)
    std_row = jnp.dot(M, std, preferred_element_type=jnp.float32)
    h = h_ref[...]
    g1 = w_ref[...] * (h - mean_row * s) / std_row + b_ref[...]
    p = jnp.where(g1 >= 0, g1, g1 * w1_ref[...])
    hp = p * dis2_ref[...]
    hp4_ref[...] = jnp.stack(
        [hp[:, 0:32], hp[:, 32:64], hp[:, 64:96], hp[:, 96:128]])


_t2b = pl.pallas_call(
    _t2b_body,
    grid=(NGRID,),
    in_specs=[
        pl.BlockSpec((NB, G), lambda i: (i, 0)),
        pl.BlockSpec((G, G), lambda i: (0, 0)),
        pl.BlockSpec((G, G), lambda i: (0, 0)),
        pl.BlockSpec((1, G), lambda i: (0, 0)),
        pl.BlockSpec((NB, 1), lambda i: (i, 0)),
        pl.BlockSpec((NB, 1), lambda i: (i, 0)),
        pl.BlockSpec((1, G), lambda i: (0, 0)),
        pl.BlockSpec((1, G), lambda i: (0, 0)),
        pl.BlockSpec((1, G), lambda i: (0, 0)),
        pl.BlockSpec((1, G), lambda i: (0, 0)),
    ],
    out_specs=[pl.BlockSpec((4, NB, 32), lambda i: (0, i, 0))],
    out_shape=[jax.ShapeDtypeStruct((4, N, 32), jnp.float32)],
)


def _t3_body(s2_ref, hp4_ref, dis2_ref, batch_ref, W2_ref, b2_ref, w1_ref,
             p_ref, S2a_ref, S2b_ref):
    i = pl.program_id(0)
    agg = jnp.concatenate(
        [s2_ref[0, k] + s2_ref[1, k] + hp4_ref[k] for k in range(4)], axis=1)
    pre = dis2_ref[...] * agg
    h2 = jnp.dot(pre, W2_ref[...], preferred_element_type=jnp.float32) + b2_ref[...]
    p = jnp.where(h2 >= 0, h2, h2 * w1_ref[...])
    p_ref[...] = p
    M = _onehot(batch_ref[...])
    dn = (((0,), (0,)), ((), ()))
    sa = lax.dot_general(M, p, dn, preferred_element_type=jnp.float32)
    sb = lax.dot_general(M, p * p, dn, preferred_element_type=jnp.float32)

    @pl.when(i == 0)
    def _():
        S2a_ref[...] = sa
        S2b_ref[...] = sb

    @pl.when(i != 0)
    def _():
        S2a_ref[...] += sa
        S2b_ref[...] += sb


_t3 = pl.pallas_call(
    _t3_body,
    grid=(NGRID,),
    in_specs=[
        pl.BlockSpec((2, 4, NB, 32), lambda i: (0, 0, i, 0)),
        pl.BlockSpec((4, NB, 32), lambda i: (0, i, 0)),
        pl.BlockSpec((NB, 1), lambda i: (i, 0)),
        pl.BlockSpec((NB, 1), lambda i: (i, 0)),
        pl.BlockSpec((G, G), lambda i: (0, 0)),
        pl.BlockSpec((1, G), lambda i: (0, 0)),
        pl.BlockSpec((1, G), lambda i: (0, 0)),
    ],
    out_specs=[
        pl.BlockSpec((NB, G), lambda i: (i, 0)),
        pl.BlockSpec((G, G), lambda i: (0, 0)),
        pl.BlockSpec((G, G), lambda i: (0, 0)),
    ],
    out_shape=[
        jax.ShapeDtypeStruct((NPAD, G), jnp.float32),
        jax.ShapeDtypeStruct((G, G), jnp.float32),
        jax.ShapeDtypeStruct((G, G), jnp.float32),
    ],
)


def _t4_body(mm_ref, acc_ref):
    i = pl.program_id(0)
    blk = mm_ref[0]

    @pl.when(i == 0)
    def _():
        acc_ref[...] = blk

    @pl.when(i != 0)
    def _():
        cur = acc_ref[...]
        acc_ref[...] = jnp.concatenate(
            [jnp.maximum(cur[0:G], blk[0:G]),
             jnp.minimum(cur[G:2 * G], blk[G:2 * G])], axis=0)


_t4 = pl.pallas_call(
    _t4_body,
    grid=(NTILES,),
    in_specs=[pl.BlockSpec((1, 2 * G, 128), lambda i: (i, 0, 0))],
    out_specs=[pl.BlockSpec((2 * G, 128), lambda i: (0, 0))],
    out_shape=[jax.ShapeDtypeStruct((2 * G, 128), jnp.float32)],
)


def _t5_body(S2a_ref, S2b_ref, cnt_ref, mm_ref, w_ref, b_ref, s_ref,
             w1_ref, w2_ref, l1W_ref, l1b_ref, l2W_ref, l2b_ref,
             l3W_ref, l3b_ref, out_ref):
    cl = jnp.maximum(cnt_ref[...], 1.0)
    invc = jnp.reshape(1.0 / cl, (G, 1))
    meanp = S2a_ref[...] * invc
    ex2 = S2b_ref[...] * invc
    s = s_ref[...]
    var = ex2 - meanp * meanp * s * (2.0 - s)
    std = jnp.sqrt(var + 1e-5)
    alpha = w_ref[...] / std
    beta = b_ref[...] - alpha * meanp * s
    pmax = mm_ref[0:G]
    pmin = mm_ref[G:2 * G]
    x1 = alpha * meanp + beta
    x2 = jnp.where(alpha >= 0, alpha * pmax, alpha * pmin) + beta
    z = jnp.concatenate([x1, x2], axis=1)
    z = jnp.dot(z, l1W_ref[...], preferred_element_type=jnp.float32) + l1b_ref[...]
    z = jnp.where(z >= 0, z, z * w1_ref[...])
    z = jnp.dot(z, l2W_ref[...], preferred_element_type=jnp.float32) + l2b_ref[...]
    z = jnp.where(z >= 0, z, z * w2_ref[...])
    z = jnp.dot(z, l3W_ref[...], preferred_element_type=jnp.float32) + l3b_ref[...]
    out_ref[...] = z


_t5 = pl.pallas_call(
    _t5_body,
    grid=(1,),
    in_specs=[
        pl.BlockSpec((G, G), lambda i: (0, 0)),
        pl.BlockSpec((G, G), lambda i: (0, 0)),
        pl.BlockSpec((1, G), lambda i: (0, 0)),
        pl.BlockSpec((2 * G, 128), lambda i: (0, 0)),
        pl.BlockSpec((1, G), lambda i: (0, 0)),
        pl.BlockSpec((1, G), lambda i: (0, 0)),
        pl.BlockSpec((1, G), lambda i: (0, 0)),
        pl.BlockSpec((1, G), lambda i: (0, 0)),
        pl.BlockSpec((1, 64), lambda i: (0, 0)),
        pl.BlockSpec((2 * G, G), lambda i: (0, 0)),
        pl.BlockSpec((1, G), lambda i: (0, 0)),
        pl.BlockSpec((G, 64), lambda i: (0, 0)),
        pl.BlockSpec((1, 64), lambda i: (0, 0)),
        pl.BlockSpec((64, 1), lambda i: (0, 0)),
        pl.BlockSpec((1, 1), lambda i: (0, 0)),
    ],
    out_specs=[pl.BlockSpec((G, 1), lambda i: (0, 0))],
    out_shape=[jax.ShapeDtypeStruct((G, 1), jnp.float32)],
)


def kernel(x, edge_index, edge_attr, batch, weight1, weight2, conv1_W, conv1_b,
           conv2_W, conv2_b, lin1_W, lin1_b, lin2_W, lin2_b, lin3_W, lin3_b,
           bn1_weight, bn1_bias, bn1_mean_scale, bn2_weight, bn2_bias,
           bn2_mean_scale):
    src = edge_index[0]
    dst = edge_index[1]
    batch2d = batch[:, None]
    batch_pad = jnp.pad(batch, (0, NPAD - N))

    zeros8 = jnp.zeros((ROWS_PER_TILE, 8), jnp.float32)
    zeros32 = jnp.zeros((ROWS_PER_TILE, 32), jnp.float32)
    msg8 = jnp.concatenate(
        [jnp.ones((E, 1), jnp.float32), edge_attr[:, None],
         jnp.zeros((E, 6), jnp.float32)], axis=1)

    deg8 = _deg_kernel(msg8, dst, zeros8)
    dis1, dis2, xp = _t1(deg8, x)
    s1 = _conv1_scatter(xp, src, dst, zeros32)
    h, S1a, S1b, cnt = _t2(s1, xp, dis1, batch2d, conv1_W, conv1_b[None, :])
    (hp4,) = _t2b(h, S1a, S1b, cnt, dis2, batch2d, bn1_weight[None, :],
                  bn1_bias[None, :], bn1_mean_scale[None, :], weight1[None, :])
    s2 = _conv2_scatter(hp4, src, dst, edge_attr, zeros32)
    p, S2a, S2b = _t3(s2, hp4, dis2, batch2d, conv2_W, conv2_b[None, :],
                      weight1[None, :])
    mm = _pool_kernel(p, batch_pad)
    (mm2,) = _t4(mm.reshape(NTILES, 2 * G, 128))
    (out,) = _t5(S2a, S2b, cnt, mm2, bn2_weight[None, :], bn2_bias[None, :],
                 bn2_mean_scale[None, :], weight1[None, :], weight2[None, :],
                 lin1_W, lin1_b[None, :], lin2_W, lin2_b[None, :], lin3_W,
                 lin3_b[None, :])
    return jnp.squeeze(out)
